# baseline (speedup 1.0000x reference)
import functools

import jax
import jax.numpy as jnp
from jax.experimental import pallas as pl
from jax.experimental.pallas import tpu as pltpu
from jax.experimental.pallas import tpu_sc as plsc

_W = 128
_TC_BLOCK = 4000
_PRE_BLOCK = 2000
_MESH_SPLITS = (0, 40000, 120000, 220000, 320000, 420000, 520000, 600000)
_WORLD_SPLITS = (0, 100000, 200000)
_NSC = 32


def _project_tables(node_features, w_stack):
    n, d = node_features.shape
    k = w_stack.shape[0]
    nb = n // _PRE_BLOCK

    def body(nf_ref, w_ref, *out_refs):
        nf_blk = nf_ref[...]
        for j, out_ref in enumerate(out_refs):
            out_ref[...] = jnp.dot(nf_blk, w_ref[j],
                                   preferred_element_type=jnp.float32)

    spec = pl.BlockSpec((_PRE_BLOCK, d), lambda i: (i, 0))
    return pl.pallas_call(
        body,
        grid=(nb,),
        in_specs=[
            spec,
            pl.BlockSpec((k, d, d), lambda i: (0, 0, 0)),
        ],
        out_specs=[spec] * k,
        out_shape=[jax.ShapeDtypeStruct((n, d), jnp.float32)] * k,
    )(node_features, w_stack)


def _window_idx(s_idx, r_idx):
    e = s_idx.shape[0]
    nw = -(-e // _W)
    m = (nw - 1) * _W

    def wins(x):
        return jnp.concatenate(
            [x[:m].reshape(-1, _W), x[e - _W:].reshape(1, _W)])

    return jnp.stack([wins(s_idx), wins(r_idx)], axis=1)


def _sc_gather_add(table_p, table_q, idx, e):
    nw = idx.shape[0]
    d = table_p.shape[1]
    per = -(-nw // _NSC)
    nt = 3 * (-(-per // 3))
    mesh = plsc.VectorSubcoreMesh(core_axis_name="c", subcore_axis_name="s")

    scratch = ([pltpu.SemaphoreType.DMA] * 6
               + [pltpu.VMEM((2, _W), jnp.int32)] * 3
               + [pltpu.VMEM((_W, d), jnp.float32)] * 6)

    @functools.partial(
        pl.kernel,
        out_type=jax.ShapeDtypeStruct((e, d), jnp.float32),
        mesh=mesh,
        scratch_types=scratch,
    )
    def gather_add_kernel(tp_hbm, tq_hbm, i_hbm, o_hbm,
                          g0, g1, g2, w0, w1, w2,
                          ib0, ib1, ib2, s0, s1, s2, r0, r1, r2):
        gsem = (g0, g1, g2)
        wsem = (w0, w1, w2)
        ib = (ib0, ib1, ib2)
        sb = (s0, s1, s2)
        rb = (r0, r1, r2)
        wid = jax.lax.axis_index("s") * 2 + jax.lax.axis_index("c")
        base = wid * per
        dummy = tp_hbm.at[pl.ds(0, _W), :]

        def issue(k, b, wait_write):
            w = base + k

            @pl.when(jnp.logical_and(w < nw, k < nt))
            def _():
                if wait_write:
                    pltpu.make_async_copy(dummy, sb[b], wsem[b]).wait()
                pltpu.sync_copy(i_hbm.at[w], ib[b])
                pltpu.async_copy(tp_hbm.at[ib[b].at[0]], sb[b], gsem[b])
                pltpu.async_copy(tq_hbm.at[ib[b].at[1]], rb[b], gsem[b])

        def complete(k, b):
            w = base + k

            @pl.when(w < nw)
            def _():
                pltpu.make_async_copy(dummy, sb[b], gsem[b]).wait()
                pltpu.make_async_copy(dummy, rb[b], gsem[b]).wait()

                @pl.loop(0, _W)
                def _(rr):
                    for c in range(0, d, 16):
                        slc = (pl.ds(rr, 1), pl.ds(c, 16))
                        sb[b].at[*slc][...] = (sb[b].at[*slc][...]
                                               + rb[b].at[*slc][...])

                row = jnp.minimum(w * _W, e - _W)
                pltpu.async_copy(sb[b], o_hbm.at[pl.ds(row, _W), :],
                                 wsem[b])

        for b in range(3):
            issue(b, b, wait_write=False)

        @pl.loop(0, nt // 3)
        def _(t):
            for b in range(3):
                k = t * 3 + b
                complete(k, b)
                issue(k + 3, b, wait_write=True)

        for b in range(3):
            @pl.when(base + b < nw)
            def _():
                pltpu.make_async_copy(dummy, sb[b], wsem[b]).wait()

    return gather_add_kernel(table_p, table_q, idx)


def _edge_mlp(summed, ef, w1c, b1, w2, b2, g, beta,
              rows_total, row_off, carry=None):
    rows = summed.shape[0]
    d = ef.shape[1]
    nb = rows // _TC_BLOCK
    off = row_off // _TC_BLOCK

    def compute(sum_ref, ef_ref, w1c_ref, b1_ref, w2_ref, b2_ref,
                g_ref, beta_ref, out_ref):
        ef_blk = ef_ref[...]
        h = sum_ref[...] + b1_ref[...]
        h = h + jnp.dot(ef_blk, w1c_ref[...],
                        preferred_element_type=jnp.float32)
        h = jnp.maximum(h, 0.0)
        y = jnp.dot(h, w2_ref[...], preferred_element_type=jnp.float32)
        y = y + b2_ref[...]
        mu = jnp.mean(y, axis=-1, keepdims=True)
        yc = y - mu
        var = jnp.mean(yc * yc, axis=-1, keepdims=True)
        out_ref[...] = (ef_blk
                        + yc * jax.lax.rsqrt(var + 1e-5) * g_ref[...]
                        + beta_ref[...])

    data_specs = [
        pl.BlockSpec((_TC_BLOCK, d), lambda i: (i, 0)),
        pl.BlockSpec((_TC_BLOCK, d), lambda i: (off + i, 0)),
        pl.BlockSpec((d, d), lambda i: (0, 0)),
        pl.BlockSpec((1, d), lambda i: (0, 0)),
        pl.BlockSpec((d, d), lambda i: (0, 0)),
        pl.BlockSpec((1, d), lambda i: (0, 0)),
        pl.BlockSpec((1, d), lambda i: (0, 0)),
        pl.BlockSpec((1, d), lambda i: (0, 0)),
    ]
    out_spec = pl.BlockSpec((_TC_BLOCK, d), lambda i: (off + i, 0))
    out_shape = jax.ShapeDtypeStruct((rows_total, d), jnp.float32)
    data = (summed, ef, w1c, b1, w2, b2, g, beta)

    if carry is None:
        return pl.pallas_call(
            compute,
            grid=(nb,),
            in_specs=data_specs,
            out_specs=out_spec,
            out_shape=out_shape,
        )(*data)

    def body(carry_ref, *refs):
        del carry_ref
        compute(*refs)

    return pl.pallas_call(
        body,
        grid=(nb,),
        in_specs=[pl.BlockSpec(memory_space=pl.MemorySpace.ANY)]
        + data_specs,
        out_specs=out_spec,
        out_shape=out_shape,
        input_output_aliases={0: 0},
    )(carry, *data)


def kernel(node_features, mesh_senders, mesh_receivers, mesh_edge_features,
           world_senders, world_receivers, world_edge_features,
           W1_mesh, b1_mesh, W2_mesh, b2_mesh, g_mesh, beta_mesh,
           W1_world, b1_world, W2_world, b2_world, g_world, beta_world):
    n, d = node_features.shape
    e_mesh = mesh_senders.shape[0]
    e_world = world_senders.shape[0]

    p_world, q_world, p_mesh, q_mesh = _project_tables(
        node_features, jnp.stack([W1_world[:d], W1_world[d:2 * d],
                                  W1_mesh[:d], W1_mesh[d:2 * d]]))

    ms = mesh_senders.astype(jnp.int32)
    mr = mesh_receivers.astype(jnp.int32)

    ws = world_senders.astype(jnp.int32)
    wr = world_receivers.astype(jnp.int32)
    world_args = (W1_world[2 * d:], b1_world.reshape(1, d), W2_world,
                  b2_world.reshape(1, d), g_world.reshape(1, d),
                  beta_world.reshape(1, d))
    world_out = None
    for lo, hi in zip(_WORLD_SPLITS[:-1], _WORLD_SPLITS[1:]):
        sum_c = _sc_gather_add(
            p_world, q_world, _window_idx(ws[lo:hi], wr[lo:hi]), hi - lo)
        world_out = _edge_mlp(
            sum_c, world_edge_features, *world_args,
            e_world, lo, carry=world_out)

    mesh_args = (W1_mesh[2 * d:], b1_mesh.reshape(1, d), W2_mesh,
                 b2_mesh.reshape(1, d), g_mesh.reshape(1, d),
                 beta_mesh.reshape(1, d))
    mesh_out = None
    for lo, hi in zip(_MESH_SPLITS[:-1], _MESH_SPLITS[1:]):
        sum_c = _sc_gather_add(
            p_mesh, q_mesh, _window_idx(ms[lo:hi], mr[lo:hi]), hi - lo)
        mesh_out = _edge_mlp(
            sum_c, mesh_edge_features, *mesh_args,
            e_mesh, lo, carry=mesh_out)

    return (mesh_out, world_out)


# speedup vs baseline: 1.0165x; 1.0165x over previous
import functools

import jax
import jax.numpy as jnp
from jax.experimental import pallas as pl
from jax.experimental.pallas import tpu as pltpu
from jax.experimental.pallas import tpu_sc as plsc

_W = 128
_TC_BLOCK = 4000
_PRE_BLOCK = 2000
_MESH_SPLITS = (0, 160000, 320000, 480000, 600000)
_WORLD_SPLITS = (0, 200000)
_NSC = 32


def _project_tables(node_features, w_stack):
    n, d = node_features.shape
    k = w_stack.shape[0]
    nb = n // _PRE_BLOCK

    def body(nf_ref, w_ref, *out_refs):
        nf_blk = nf_ref[...]
        for j, out_ref in enumerate(out_refs):
            out_ref[...] = jnp.dot(nf_blk, w_ref[j],
                                   preferred_element_type=jnp.float32)

    spec = pl.BlockSpec((_PRE_BLOCK, d), lambda i: (i, 0))
    return pl.pallas_call(
        body,
        grid=(nb,),
        in_specs=[
            spec,
            pl.BlockSpec((k, d, d), lambda i: (0, 0, 0)),
        ],
        out_specs=[spec] * k,
        out_shape=[jax.ShapeDtypeStruct((n, d), jnp.float32)] * k,
    )(node_features, w_stack)


def _window_idx(s_idx, r_idx):
    e = s_idx.shape[0]
    nw = -(-e // _W)
    m = (nw - 1) * _W

    def wins(x):
        return jnp.concatenate(
            [x[:m].reshape(-1, _W), x[e - _W:].reshape(1, _W)])

    return jnp.stack([wins(s_idx), wins(r_idx)], axis=1)


def _sc_gather_add(table_p, table_q, idx, e):
    nw = idx.shape[0]
    d = table_p.shape[1]
    per = -(-nw // _NSC)
    nt = 3 * (-(-per // 3))
    mesh = plsc.VectorSubcoreMesh(core_axis_name="c", subcore_axis_name="s")

    scratch = ([pltpu.SemaphoreType.DMA] * 6
               + [pltpu.VMEM((2, _W), jnp.int32)] * 3
               + [pltpu.VMEM((_W, d), jnp.float32)] * 6)

    @functools.partial(
        pl.kernel,
        out_type=jax.ShapeDtypeStruct((e, d), jnp.float32),
        mesh=mesh,
        scratch_types=scratch,
    )
    def gather_add_kernel(tp_hbm, tq_hbm, i_hbm, o_hbm,
                          g0, g1, g2, w0, w1, w2,
                          ib0, ib1, ib2, s0, s1, s2, r0, r1, r2):
        gsem = (g0, g1, g2)
        wsem = (w0, w1, w2)
        ib = (ib0, ib1, ib2)
        sb = (s0, s1, s2)
        rb = (r0, r1, r2)
        wid = jax.lax.axis_index("s") * 2 + jax.lax.axis_index("c")
        base = wid * per
        dummy = tp_hbm.at[pl.ds(0, _W), :]

        def issue(k, b, wait_write):
            w = base + k

            @pl.when(jnp.logical_and(w < nw, k < nt))
            def _():
                if wait_write:
                    pltpu.make_async_copy(dummy, sb[b], wsem[b]).wait()
                pltpu.sync_copy(i_hbm.at[w], ib[b])
                pltpu.async_copy(tp_hbm.at[ib[b].at[0]], sb[b], gsem[b])
                pltpu.async_copy(tq_hbm.at[ib[b].at[1]], rb[b], gsem[b])

        def complete(k, b):
            w = base + k

            @pl.when(w < nw)
            def _():
                pltpu.make_async_copy(dummy, sb[b], gsem[b]).wait()
                pltpu.make_async_copy(dummy, rb[b], gsem[b]).wait()

                @pl.loop(0, _W)
                def _(rr):
                    for c in range(0, d, 16):
                        slc = (pl.ds(rr, 1), pl.ds(c, 16))
                        sb[b].at[*slc][...] = (sb[b].at[*slc][...]
                                               + rb[b].at[*slc][...])

                row = jnp.minimum(w * _W, e - _W)
                pltpu.async_copy(sb[b], o_hbm.at[pl.ds(row, _W), :],
                                 wsem[b])

        for b in range(3):
            issue(b, b, wait_write=False)

        @pl.loop(0, nt // 3)
        def _(t):
            for b in range(3):
                k = t * 3 + b
                complete(k, b)
                issue(k + 3, b, wait_write=True)

        for b in range(3):
            @pl.when(base + b < nw)
            def _():
                pltpu.make_async_copy(dummy, sb[b], wsem[b]).wait()

    return gather_add_kernel(table_p, table_q, idx)


def _edge_mlp(summed, ef, w1c, b1, w2, b2, g, beta,
              rows_total, row_off, carry=None):
    rows = summed.shape[0]
    d = ef.shape[1]
    nb = rows // _TC_BLOCK
    off = row_off // _TC_BLOCK

    def compute(sum_ref, ef_ref, w1c_ref, b1_ref, w2_ref, b2_ref,
                g_ref, beta_ref, out_ref):
        ef_blk = ef_ref[...]
        h = sum_ref[...] + b1_ref[...]
        h = h + jnp.dot(ef_blk, w1c_ref[...],
                        preferred_element_type=jnp.float32)
        h = jnp.maximum(h, 0.0)
        y = jnp.dot(h, w2_ref[...], preferred_element_type=jnp.float32)
        y = y + b2_ref[...]
        mu = jnp.mean(y, axis=-1, keepdims=True)
        yc = y - mu
        var = jnp.mean(yc * yc, axis=-1, keepdims=True)
        out_ref[...] = (ef_blk
                        + yc * jax.lax.rsqrt(var + 1e-5) * g_ref[...]
                        + beta_ref[...])

    data_specs = [
        pl.BlockSpec((_TC_BLOCK, d), lambda i: (i, 0)),
        pl.BlockSpec((_TC_BLOCK, d), lambda i: (off + i, 0)),
        pl.BlockSpec((d, d), lambda i: (0, 0)),
        pl.BlockSpec((1, d), lambda i: (0, 0)),
        pl.BlockSpec((d, d), lambda i: (0, 0)),
        pl.BlockSpec((1, d), lambda i: (0, 0)),
        pl.BlockSpec((1, d), lambda i: (0, 0)),
        pl.BlockSpec((1, d), lambda i: (0, 0)),
    ]
    out_spec = pl.BlockSpec((_TC_BLOCK, d), lambda i: (off + i, 0))
    out_shape = jax.ShapeDtypeStruct((rows_total, d), jnp.float32)
    data = (summed, ef, w1c, b1, w2, b2, g, beta)

    if carry is None:
        return pl.pallas_call(
            compute,
            grid=(nb,),
            in_specs=data_specs,
            out_specs=out_spec,
            out_shape=out_shape,
        )(*data)

    def body(carry_ref, *refs):
        del carry_ref
        compute(*refs)

    return pl.pallas_call(
        body,
        grid=(nb,),
        in_specs=[pl.BlockSpec(memory_space=pl.MemorySpace.ANY)]
        + data_specs,
        out_specs=out_spec,
        out_shape=out_shape,
        input_output_aliases={0: 0},
    )(carry, *data)


def kernel(node_features, mesh_senders, mesh_receivers, mesh_edge_features,
           world_senders, world_receivers, world_edge_features,
           W1_mesh, b1_mesh, W2_mesh, b2_mesh, g_mesh, beta_mesh,
           W1_world, b1_world, W2_world, b2_world, g_world, beta_world):
    n, d = node_features.shape
    e_mesh = mesh_senders.shape[0]
    e_world = world_senders.shape[0]

    p_world, q_world, p_mesh, q_mesh = _project_tables(
        node_features, jnp.stack([W1_world[:d], W1_world[d:2 * d],
                                  W1_mesh[:d], W1_mesh[d:2 * d]]))

    ms = mesh_senders.astype(jnp.int32)
    mr = mesh_receivers.astype(jnp.int32)

    ws = world_senders.astype(jnp.int32)
    wr = world_receivers.astype(jnp.int32)
    world_args = (W1_world[2 * d:], b1_world.reshape(1, d), W2_world,
                  b2_world.reshape(1, d), g_world.reshape(1, d),
                  beta_world.reshape(1, d))
    world_out = None
    for lo, hi in zip(_WORLD_SPLITS[:-1], _WORLD_SPLITS[1:]):
        sum_c = _sc_gather_add(
            p_world, q_world, _window_idx(ws[lo:hi], wr[lo:hi]), hi - lo)
        world_out = _edge_mlp(
            sum_c, world_edge_features, *world_args,
            e_world, lo, carry=world_out)

    mesh_args = (W1_mesh[2 * d:], b1_mesh.reshape(1, d), W2_mesh,
                 b2_mesh.reshape(1, d), g_mesh.reshape(1, d),
                 beta_mesh.reshape(1, d))
    mesh_out = None
    for lo, hi in zip(_MESH_SPLITS[:-1], _MESH_SPLITS[1:]):
        sum_c = _sc_gather_add(
            p_mesh, q_mesh, _window_idx(ms[lo:hi], mr[lo:hi]), hi - lo)
        mesh_out = _edge_mlp(
            sum_c, mesh_edge_features, *mesh_args,
            e_mesh, lo, carry=mesh_out)

    return (mesh_out, world_out)


# speedup vs baseline: 1.0301x; 1.0134x over previous
import functools

import jax
import jax.numpy as jnp
from jax.experimental import pallas as pl
from jax.experimental.pallas import tpu as pltpu
from jax.experimental.pallas import tpu_sc as plsc

_W = 128
_TC_BLOCK = 4000
_PRE_BLOCK = 2000
_MESH_SPLITS = (0, 240000, 480000, 600000)
_NSC = 32


def _project_tables(node_features, w_stack):
    n, d = node_features.shape
    k = w_stack.shape[0]
    nb = n // _PRE_BLOCK

    def body(nf_ref, w_ref, *out_refs):
        nf_blk = nf_ref[...]
        for j, out_ref in enumerate(out_refs):
            out_ref[...] = jnp.dot(nf_blk, w_ref[j],
                                   preferred_element_type=jnp.float32)

    spec = pl.BlockSpec((_PRE_BLOCK, d), lambda i: (i, 0))
    return pl.pallas_call(
        body,
        grid=(nb,),
        in_specs=[
            spec,
            pl.BlockSpec((k, d, d), lambda i: (0, 0, 0)),
        ],
        out_specs=[spec] * k,
        out_shape=[jax.ShapeDtypeStruct((n, d), jnp.float32)] * k,
    )(node_features, w_stack)


def _window_idx(s_idx, r_idx):
    e = s_idx.shape[0]
    nw = -(-e // _W)
    m = (nw - 1) * _W

    def wins(x):
        return jnp.concatenate(
            [x[:m].reshape(-1, _W), x[e - _W:].reshape(1, _W)])

    return jnp.stack([wins(s_idx), wins(r_idx)], axis=1)


def _sc_gather_add(table_p, table_q, idx, e):
    nw = idx.shape[0]
    d = table_p.shape[1]
    per = -(-nw // _NSC)
    nt = 3 * (-(-per // 3))
    mesh = plsc.VectorSubcoreMesh(core_axis_name="c", subcore_axis_name="s")

    scratch = ([pltpu.SemaphoreType.DMA] * 6
               + [pltpu.VMEM((2, _W), jnp.int32)] * 3
               + [pltpu.VMEM((_W, d), jnp.float32)] * 6)

    @functools.partial(
        pl.kernel,
        out_type=jax.ShapeDtypeStruct((e, d), jnp.float32),
        mesh=mesh,
        scratch_types=scratch,
    )
    def gather_add_kernel(tp_hbm, tq_hbm, i_hbm, o_hbm,
                          g0, g1, g2, w0, w1, w2,
                          ib0, ib1, ib2, s0, s1, s2, r0, r1, r2):
        gsem = (g0, g1, g2)
        wsem = (w0, w1, w2)
        ib = (ib0, ib1, ib2)
        sb = (s0, s1, s2)
        rb = (r0, r1, r2)
        wid = jax.lax.axis_index("s") * 2 + jax.lax.axis_index("c")
        base = wid * per
        dummy = tp_hbm.at[pl.ds(0, _W), :]

        def issue(k, b, wait_write):
            w = base + k

            @pl.when(jnp.logical_and(w < nw, k < nt))
            def _():
                if wait_write:
                    pltpu.make_async_copy(dummy, sb[b], wsem[b]).wait()
                pltpu.sync_copy(i_hbm.at[w], ib[b])
                pltpu.async_copy(tp_hbm.at[ib[b].at[0]], sb[b], gsem[b])
                pltpu.async_copy(tq_hbm.at[ib[b].at[1]], rb[b], gsem[b])

        def complete(k, b):
            w = base + k

            @pl.when(w < nw)
            def _():
                pltpu.make_async_copy(dummy, sb[b], gsem[b]).wait()
                pltpu.make_async_copy(dummy, rb[b], gsem[b]).wait()

                @pl.loop(0, _W)
                def _(rr):
                    for c in range(0, d, 16):
                        slc = (pl.ds(rr, 1), pl.ds(c, 16))
                        sb[b].at[*slc][...] = (sb[b].at[*slc][...]
                                               + rb[b].at[*slc][...])

                row = jnp.minimum(w * _W, e - _W)
                pltpu.async_copy(sb[b], o_hbm.at[pl.ds(row, _W), :],
                                 wsem[b])

        for b in range(3):
            issue(b, b, wait_write=False)

        @pl.loop(0, nt // 3)
        def _(t):
            for b in range(3):
                k = t * 3 + b
                complete(k, b)
                issue(k + 3, b, wait_write=True)

        for b in range(3):
            @pl.when(base + b < nw)
            def _():
                pltpu.make_async_copy(dummy, sb[b], wsem[b]).wait()

    return gather_add_kernel(table_p, table_q, idx)


def _edge_mlp(summed, ef, w1c, b1, w2, b2, g, beta,
              rows_total, row_off, carry=None):
    rows = summed.shape[0]
    d = ef.shape[1]
    nb = rows // _TC_BLOCK
    off = row_off // _TC_BLOCK

    def compute(sum_ref, ef_ref, w1c_ref, b1_ref, w2_ref, b2_ref,
                g_ref, beta_ref, out_ref):
        ef_blk = ef_ref[...]
        h = sum_ref[...] + b1_ref[...]
        h = h + jnp.dot(ef_blk, w1c_ref[...],
                        preferred_element_type=jnp.float32)
        h = jnp.maximum(h, 0.0)
        y = jnp.dot(h, w2_ref[...], preferred_element_type=jnp.float32)
        y = y + b2_ref[...]
        mu = jnp.mean(y, axis=-1, keepdims=True)
        yc = y - mu
        var = jnp.mean(yc * yc, axis=-1, keepdims=True)
        out_ref[...] = (ef_blk
                        + yc * jax.lax.rsqrt(var + 1e-5) * g_ref[...]
                        + beta_ref[...])

    data_specs = [
        pl.BlockSpec((_TC_BLOCK, d), lambda i: (i, 0)),
        pl.BlockSpec((_TC_BLOCK, d), lambda i: (off + i, 0)),
        pl.BlockSpec((d, d), lambda i: (0, 0)),
        pl.BlockSpec((1, d), lambda i: (0, 0)),
        pl.BlockSpec((d, d), lambda i: (0, 0)),
        pl.BlockSpec((1, d), lambda i: (0, 0)),
        pl.BlockSpec((1, d), lambda i: (0, 0)),
        pl.BlockSpec((1, d), lambda i: (0, 0)),
    ]
    out_spec = pl.BlockSpec((_TC_BLOCK, d), lambda i: (off + i, 0))
    out_shape = jax.ShapeDtypeStruct((rows_total, d), jnp.float32)
    data = (summed, ef, w1c, b1, w2, b2, g, beta)

    if carry is None:
        return pl.pallas_call(
            compute,
            grid=(nb,),
            in_specs=data_specs,
            out_specs=out_spec,
            out_shape=out_shape,
        )(*data)

    def body(carry_ref, *refs):
        del carry_ref
        compute(*refs)

    return pl.pallas_call(
        body,
        grid=(nb,),
        in_specs=[pl.BlockSpec(memory_space=pl.MemorySpace.ANY)]
        + data_specs,
        out_specs=out_spec,
        out_shape=out_shape,
        input_output_aliases={0: 0},
    )(carry, *data)


def kernel(node_features, mesh_senders, mesh_receivers, mesh_edge_features,
           world_senders, world_receivers, world_edge_features,
           W1_mesh, b1_mesh, W2_mesh, b2_mesh, g_mesh, beta_mesh,
           W1_world, b1_world, W2_world, b2_world, g_world, beta_world):
    n, d = node_features.shape
    e_mesh = mesh_senders.shape[0]
    e_world = world_senders.shape[0]

    p_world, q_world, p_mesh, q_mesh = _project_tables(
        node_features, jnp.stack([W1_world[:d], W1_world[d:2 * d],
                                  W1_mesh[:d], W1_mesh[d:2 * d]]))

    ms = mesh_senders.astype(jnp.int32)
    mr = mesh_receivers.astype(jnp.int32)

    sum_world = _sc_gather_add(
        p_world, q_world,
        _window_idx(world_senders.astype(jnp.int32),
                    world_receivers.astype(jnp.int32)),
        e_world)
    world_out = _edge_mlp(
        sum_world, world_edge_features, W1_world[2 * d:],
        b1_world.reshape(1, d), W2_world, b2_world.reshape(1, d),
        g_world.reshape(1, d), beta_world.reshape(1, d),
        e_world, 0)

    mesh_args = (W1_mesh[2 * d:], b1_mesh.reshape(1, d), W2_mesh,
                 b2_mesh.reshape(1, d), g_mesh.reshape(1, d),
                 beta_mesh.reshape(1, d))
    mesh_out = None
    for lo, hi in zip(_MESH_SPLITS[:-1], _MESH_SPLITS[1:]):
        sum_c = _sc_gather_add(
            p_mesh, q_mesh, _window_idx(ms[lo:hi], mr[lo:hi]), hi - lo)
        mesh_out = _edge_mlp(
            sum_c, mesh_edge_features, *mesh_args,
            e_mesh, lo, carry=mesh_out)

    return (mesh_out, world_out)


# speedup vs baseline: 1.0404x; 1.0100x over previous
import functools

import jax
import jax.numpy as jnp
from jax.experimental import pallas as pl
from jax.experimental.pallas import tpu as pltpu
from jax.experimental.pallas import tpu_sc as plsc

_W = 128
_TC_BLOCK = 4000
_PRE_BLOCK = 2000
_MESH_SPLITS = (0, 240000, 480000, 600000)
_NSC = 32


def _project_tables(node_features, w_stack):
    n, d = node_features.shape
    k = w_stack.shape[0]
    nb = n // _PRE_BLOCK

    def body(nf_ref, w_ref, *out_refs):
        nf_blk = nf_ref[...]
        for j, out_ref in enumerate(out_refs):
            out_ref[...] = jnp.dot(nf_blk, w_ref[j],
                                   preferred_element_type=jnp.float32)

    spec = pl.BlockSpec((_PRE_BLOCK, d), lambda i: (i, 0))
    return pl.pallas_call(
        body,
        grid=(nb,),
        in_specs=[
            spec,
            pl.BlockSpec((k, d, d), lambda i: (0, 0, 0)),
        ],
        out_specs=[spec] * k,
        out_shape=[jax.ShapeDtypeStruct((n, d), jnp.float32)] * k,
    )(node_features, w_stack)


def _window_idx(s_idx, r_idx):
    e = s_idx.shape[0]
    nw = -(-e // _W)
    m = (nw - 1) * _W

    def wins(x):
        return jnp.concatenate(
            [x[:m].reshape(-1, _W), x[e - _W:].reshape(1, _W)])

    return jnp.stack([wins(s_idx), wins(r_idx)], axis=1)


def _sc_gather_add(table_p, table_q, idx, e):
    nw = idx.shape[0]
    d = table_p.shape[1]
    per = -(-nw // _NSC)
    nt = 3 * (-(-per // 3))
    mesh = plsc.VectorSubcoreMesh(core_axis_name="c", subcore_axis_name="s")

    scratch = ([pltpu.SemaphoreType.DMA] * 6
               + [pltpu.VMEM((2, _W), jnp.int32)] * 3
               + [pltpu.VMEM((_W, d), jnp.float32)] * 6)

    @functools.partial(
        pl.kernel,
        out_type=jax.ShapeDtypeStruct((e, d), jnp.float32),
        mesh=mesh,
        scratch_types=scratch,
    )
    def gather_add_kernel(tp_hbm, tq_hbm, i_hbm, o_hbm,
                          g0, g1, g2, w0, w1, w2,
                          ib0, ib1, ib2, s0, s1, s2, r0, r1, r2):
        gsem = (g0, g1, g2)
        wsem = (w0, w1, w2)
        ib = (ib0, ib1, ib2)
        sb = (s0, s1, s2)
        rb = (r0, r1, r2)
        wid = jax.lax.axis_index("s") * 2 + jax.lax.axis_index("c")
        base = wid * per
        dummy = tp_hbm.at[pl.ds(0, _W), :]

        def issue(k, b, wait_write):
            w = base + k

            @pl.when(jnp.logical_and(w < nw, k < per))
            def _():
                if wait_write:
                    pltpu.make_async_copy(dummy, sb[b], wsem[b]).wait()
                pltpu.sync_copy(i_hbm.at[w], ib[b])
                pltpu.async_copy(tp_hbm.at[ib[b].at[0]], sb[b], gsem[b])
                pltpu.async_copy(tq_hbm.at[ib[b].at[1]], rb[b], gsem[b])

        def complete(k, b):
            w = base + k

            @pl.when(jnp.logical_and(w < nw, k < per))
            def _():
                pltpu.make_async_copy(dummy, sb[b], gsem[b]).wait()
                pltpu.make_async_copy(dummy, rb[b], gsem[b]).wait()

                @pl.loop(0, _W)
                def _(rr):
                    for c in range(0, d, 16):
                        slc = (pl.ds(rr, 1), pl.ds(c, 16))
                        sb[b].at[*slc][...] = (sb[b].at[*slc][...]
                                               + rb[b].at[*slc][...])

                row = jnp.minimum(w * _W, e - _W)
                pltpu.async_copy(sb[b], o_hbm.at[pl.ds(row, _W), :],
                                 wsem[b])

        for b in range(3):
            issue(b, b, wait_write=False)

        @pl.loop(0, nt // 3)
        def _(t):
            for b in range(3):
                k = t * 3 + b
                complete(k, b)
                issue(k + 3, b, wait_write=True)

        for b in range(3):
            @pl.when(base + b < nw)
            def _():
                pltpu.make_async_copy(dummy, sb[b], wsem[b]).wait()

    return gather_add_kernel(table_p, table_q, idx)


def _edge_mlp(summed, ef, w1c, b1, w2, b2, g, beta,
              rows_total, row_off, carry=None):
    rows = summed.shape[0]
    d = ef.shape[1]
    nb = rows // _TC_BLOCK
    off = row_off // _TC_BLOCK

    def compute(sum_ref, ef_ref, w1c_ref, b1_ref, w2_ref, b2_ref,
                g_ref, beta_ref, out_ref):
        ef_blk = ef_ref[...]
        h = sum_ref[...] + b1_ref[...]
        h = h + jnp.dot(ef_blk, w1c_ref[...],
                        preferred_element_type=jnp.float32)
        h = jnp.maximum(h, 0.0)
        y = jnp.dot(h, w2_ref[...], preferred_element_type=jnp.float32)
        y = y + b2_ref[...]
        mu = jnp.mean(y, axis=-1, keepdims=True)
        yc = y - mu
        var = jnp.mean(yc * yc, axis=-1, keepdims=True)
        out_ref[...] = (ef_blk
                        + yc * jax.lax.rsqrt(var + 1e-5) * g_ref[...]
                        + beta_ref[...])

    data_specs = [
        pl.BlockSpec((_TC_BLOCK, d), lambda i: (i, 0)),
        pl.BlockSpec((_TC_BLOCK, d), lambda i: (off + i, 0)),
        pl.BlockSpec((d, d), lambda i: (0, 0)),
        pl.BlockSpec((1, d), lambda i: (0, 0)),
        pl.BlockSpec((d, d), lambda i: (0, 0)),
        pl.BlockSpec((1, d), lambda i: (0, 0)),
        pl.BlockSpec((1, d), lambda i: (0, 0)),
        pl.BlockSpec((1, d), lambda i: (0, 0)),
    ]
    out_spec = pl.BlockSpec((_TC_BLOCK, d), lambda i: (off + i, 0))
    out_shape = jax.ShapeDtypeStruct((rows_total, d), jnp.float32)
    data = (summed, ef, w1c, b1, w2, b2, g, beta)

    if carry is None:
        return pl.pallas_call(
            compute,
            grid=(nb,),
            in_specs=data_specs,
            out_specs=out_spec,
            out_shape=out_shape,
        )(*data)

    def body(carry_ref, *refs):
        del carry_ref
        compute(*refs)

    return pl.pallas_call(
        body,
        grid=(nb,),
        in_specs=[pl.BlockSpec(memory_space=pl.MemorySpace.ANY)]
        + data_specs,
        out_specs=out_spec,
        out_shape=out_shape,
        input_output_aliases={0: 0},
    )(carry, *data)


def kernel(node_features, mesh_senders, mesh_receivers, mesh_edge_features,
           world_senders, world_receivers, world_edge_features,
           W1_mesh, b1_mesh, W2_mesh, b2_mesh, g_mesh, beta_mesh,
           W1_world, b1_world, W2_world, b2_world, g_world, beta_world):
    n, d = node_features.shape
    e_mesh = mesh_senders.shape[0]
    e_world = world_senders.shape[0]

    p_world, q_world, p_mesh, q_mesh = _project_tables(
        node_features, jnp.stack([W1_world[:d], W1_world[d:2 * d],
                                  W1_mesh[:d], W1_mesh[d:2 * d]]))

    ms = mesh_senders.astype(jnp.int32)
    mr = mesh_receivers.astype(jnp.int32)

    sum_world = _sc_gather_add(
        p_world, q_world,
        _window_idx(world_senders.astype(jnp.int32),
                    world_receivers.astype(jnp.int32)),
        e_world)
    world_out = _edge_mlp(
        sum_world, world_edge_features, W1_world[2 * d:],
        b1_world.reshape(1, d), W2_world, b2_world.reshape(1, d),
        g_world.reshape(1, d), beta_world.reshape(1, d),
        e_world, 0)

    mesh_args = (W1_mesh[2 * d:], b1_mesh.reshape(1, d), W2_mesh,
                 b2_mesh.reshape(1, d), g_mesh.reshape(1, d),
                 beta_mesh.reshape(1, d))
    mesh_out = None
    for lo, hi in zip(_MESH_SPLITS[:-1], _MESH_SPLITS[1:]):
        sum_c = _sc_gather_add(
            p_mesh, q_mesh, _window_idx(ms[lo:hi], mr[lo:hi]), hi - lo)
        mesh_out = _edge_mlp(
            sum_c, mesh_edge_features, *mesh_args,
            e_mesh, lo, carry=mesh_out)

    return (mesh_out, world_out)
